# baseline (speedup 1.0000x reference)
import functools

import jax
import jax.numpy as jnp
from jax import lax
from jax.experimental import pallas as pl
from jax.experimental.pallas import tpu as pltpu
from jax.experimental.pallas import tpu_sc as plsc

N = 10000
DEG = 16
F = 128
H = 4
NW = 32
NPW = 320
N_PAD = NW * NPW
BLK = 8
NBLK = NPW // BLK
SVC = 16
AW = 128
MB = 256


def _tc_pre_body(f_ref, u_ref, c_ref, out_ref):
    out_ref[...] = (
        jnp.dot(f_ref[...], u_ref[...], preferred_element_type=jnp.float32)
        + c_ref[...]
    )


def _tc_mix_body(gf_ref, a_ref, xs_ref, wt_ref, wb_ref, out_ref):
    af = a_ref[...]
    xs = xs_ref[...]
    accs = [af[:, 64 + h][:, None] * xs for h in range(H)]
    for j in range(DEG):
        g = gf_ref[:, j * F:(j + 1) * F]
        for h in range(H):
            accs[h] = accs[h] + af[:, h * 16 + j][:, None] * g
    for h in range(H):
        sl = slice(h * F, (h + 1) * F)
        out_ref[:, sl] = (
            jnp.dot(accs[h], wt_ref[h], preferred_element_type=jnp.float32)
            + wb_ref[:, sl]
        )


def _splat(v, j):
    return v.at[jnp.full((16,), j, jnp.int32)].get(mode="promise_in_bounds")


def _perm(v, p):
    return v.at[p].get(mode="promise_in_bounds")


def _xor_perms():
    lanes = lax.iota(jnp.int32, 16)
    return [lanes ^ k for k in (1, 2, 4, 8)]


def _lanemax(v):
    for p in _xor_perms():
        v = jnp.maximum(v, _perm(v, p))
    return v


def _lanesum(v):
    for p in _xor_perms():
        v = v + _perm(v, p)
    return v


def _sc_alpha_body(
    adj_hbm, sv_hbm, feat_hbm, alph_hbm, gfeat_hbm,
    idx_v, sn0, sn1, ss0, ss1, ao0, ao1, gn0, gn1, nsem, ssem, asem, gsem,
    osem
):
    wid = lax.axis_index("s") * 2 + lax.axis_index("c")
    base = wid * NPW
    sns = [sn0, sn1]
    sss = [ss0, ss1]
    aos = [ao0, ao1]
    gns = [gn0, gn1]

    pltpu.sync_copy(adj_hbm.at[pl.ds(wid * NBLK, NBLK), :], idx_v)

    def issue_in(b, slot):
        pltpu.async_copy(sv_hbm.at[idx_v.at[b]], sns[slot], nsem.at[slot])
        pltpu.async_copy(
            sv_hbm.at[pl.ds(base + b * BLK, BLK), :], sss[slot], ssem.at[slot]
        )
        pltpu.async_copy(feat_hbm.at[idx_v.at[b]], gns[slot], gsem.at[slot])

    issue_in(0, 0)
    issue_in(1, 1)

    @pl.loop(0, NBLK, step=2)
    def _blk2(bi0):
        for par in (0, 1):
            b = bi0 + par
            slot = par
            sn_v, ss_v, ao_v, gn_v = sns[slot], sss[slot], aos[slot], gns[slot]
            grows = pl.ds((base + b * BLK) * DEG, BLK * DEG)
            pltpu.make_async_copy(
                feat_hbm.at[idx_v.at[b]], gn_v, gsem.at[slot]
            ).wait()
            pltpu.async_copy(gn_v, gfeat_hbm.at[grows, :], osem.at[slot])
            pltpu.make_async_copy(
                sv_hbm.at[idx_v.at[b]], sn_v, nsem.at[slot]
            ).wait()
            pltpu.make_async_copy(
                sv_hbm.at[pl.ds(base + b * BLK, BLK), :], ss_v, ssem.at[slot]
            ).wait()

            @pl.when(b >= 2)
            def _():
                pltpu.make_async_copy(
                    ao_v, alph_hbm.at[pl.ds(base + b * BLK, BLK), :],
                    asem.at[slot],
                ).wait()

            _alpha_block(sn_v, ss_v, ao_v)
            pltpu.async_copy(
                ao_v, alph_hbm.at[pl.ds(base + b * BLK, BLK), :], asem.at[slot]
            )

            @pl.when(b + 2 < NBLK)
            def _():
                pltpu.make_async_copy(
                    gn_v, gfeat_hbm.at[grows, :], osem.at[slot]
                ).wait()
                issue_in(b + 2, slot)

    for slot in (0, 1):
        b = NBLK - 2 + slot
        grows = pl.ds((base + b * BLK) * DEG, BLK * DEG)
        pltpu.make_async_copy(
            aos[slot], alph_hbm.at[pl.ds(base + b * BLK, BLK), :],
            asem.at[slot],
        ).wait()
        pltpu.make_async_copy(
            gns[slot], gfeat_hbm.at[grows, :], osem.at[slot]
        ).wait()


def _alpha_block(sn_v, ss_v, ao_v):
    lanes = lax.iota(jnp.int32, 16)

    @pl.loop(0, BLK)
    def _node(i):
        rbase = i * DEG
        svrow = ss_v[i, :]
        aselfs = []
        for h in range(H):
            col2 = jnp.full((16,), H + h, jnp.int32)
            s1s = _splat(svrow, h)
            s2s = _splat(svrow, H + h)
            s2n = plsc.load_gather(sn_v, [rbase + lanes, col2])
            en = s1s + s2n
            en = jnp.where(en > 0, en, 0.2 * en)
            ev = s1s + s2s
            ev = jnp.where(ev > 0, ev, 0.2 * ev)
            m = jnp.maximum(_lanemax(en), ev)
            pn = jnp.exp(en - m)
            pv = jnp.exp(ev - m)
            r = 1.0 / (_lanesum(pn) + pv)
            ao_v[i, pl.ds(h * 16, 16)] = pn * r
            aselfs.append(pv * r)
        comb = aselfs[0]
        for h in range(1, H):
            comb = jnp.where(lanes == h, aselfs[h], comb)
        ao_v[i, pl.ds(64, 16)] = comb


_sc_cp = pltpu.CompilerParams(
    needs_layout_passes=False, use_tc_tiling_on_sc=False
)

_sc_alpha = functools.partial(
    pl.kernel,
    compiler_params=_sc_cp,
    out_type=(
        jax.ShapeDtypeStruct((N_PAD, AW), jnp.float32),
        jax.ShapeDtypeStruct((N_PAD * DEG, F), jnp.float32),
    ),
    mesh=plsc.VectorSubcoreMesh(core_axis_name="c", subcore_axis_name="s"),
    scratch_types=[
        pltpu.VMEM((NBLK, BLK * DEG), jnp.int32),
        pltpu.VMEM((BLK * DEG, SVC), jnp.float32),
        pltpu.VMEM((BLK * DEG, SVC), jnp.float32),
        pltpu.VMEM((BLK, SVC), jnp.float32),
        pltpu.VMEM((BLK, SVC), jnp.float32),
        pltpu.VMEM((BLK, AW), jnp.float32),
        pltpu.VMEM((BLK, AW), jnp.float32),
        pltpu.VMEM((BLK * DEG, F), jnp.float32),
        pltpu.VMEM((BLK * DEG, F), jnp.float32),
        pltpu.SemaphoreType.DMA((2,)),
        pltpu.SemaphoreType.DMA((2,)),
        pltpu.SemaphoreType.DMA((2,)),
        pltpu.SemaphoreType.DMA((2,)),
        pltpu.SemaphoreType.DMA((2,)),
    ],
)(_sc_alpha_body)


def kernel(adjlist, features, W, Wb, A, Ab):
    a1 = A[:, :F]
    a2 = A[:, F:]
    u1 = jnp.einsum("ho,hof->hf", a1, W)
    u2 = jnp.einsum("ho,hof->hf", a2, W)
    c1 = jnp.sum(a1 * Wb, axis=1) + Ab
    c2 = jnp.sum(a2 * Wb, axis=1)
    u = jnp.concatenate([u1.T, u2.T, jnp.zeros((F, 8), jnp.float32)], axis=1)
    cvec = jnp.concatenate([c1, c2, jnp.zeros((8,), jnp.float32)])[None, :]
    wt = jnp.swapaxes(W, 1, 2)
    wb_flat = Wb.reshape(1, H * F)

    feat_pad = jnp.pad(features, ((0, N_PAD - N), (0, 0)))
    adj2d = jnp.pad(adjlist.astype(jnp.int32), ((0, N_PAD - N), (0, 0))).reshape(
        NW * NBLK, BLK * DEG
    )

    sv = pl.pallas_call(
        _tc_pre_body,
        grid=(N_PAD // 1024,),
        in_specs=[
            pl.BlockSpec((1024, F), lambda i: (i, 0)),
            pl.BlockSpec((F, SVC), lambda i: (0, 0)),
            pl.BlockSpec((1, SVC), lambda i: (0, 0)),
        ],
        out_specs=pl.BlockSpec((1024, SVC), lambda i: (i, 0)),
        out_shape=jax.ShapeDtypeStruct((N_PAD, SVC), jnp.float32),
    )(feat_pad, u, cvec)

    alph, gfeat = _sc_alpha(adj2d, sv, feat_pad)
    gf2 = gfeat.reshape(N_PAD, DEG * F)

    out = pl.pallas_call(
        _tc_mix_body,
        grid=(N_PAD // MB,),
        in_specs=[
            pl.BlockSpec((MB, DEG * F), lambda i: (i, 0)),
            pl.BlockSpec((MB, AW), lambda i: (i, 0)),
            pl.BlockSpec((MB, F), lambda i: (i, 0)),
            pl.BlockSpec((H, F, F), lambda i: (0, 0, 0)),
            pl.BlockSpec((1, H * F), lambda i: (0, 0)),
        ],
        out_specs=pl.BlockSpec((MB, H * F), lambda i: (i, 0)),
        out_shape=jax.ShapeDtypeStruct((N_PAD, H * F), jnp.float32),
    )(gf2, alph, feat_pad, wt, wb_flat)
    return out[:N]


# speedup vs baseline: 1.3761x; 1.3761x over previous
import dataclasses
import functools

import jax
import jax.numpy as jnp
import numpy as np
from jax import lax
from jax.experimental import pallas as pl
from jax.experimental.pallas import tpu as pltpu
from jax.experimental.pallas import tpu_sc as plsc

N = 10000
DEG = 16
F = 128
H = 4
NW = 32
NPW = 320
N_PAD = NW * NPW
BLK = 8
NBLK = NPW // BLK
C = 144
NCH = F // 16


def _tc_pre_body(f_ref, u_ref, c_ref, out_ref):
    f = f_ref[...]
    out_ref[:, :F] = f
    out_ref[:, F:] = (
        jnp.dot(f, u_ref[...], preferred_element_type=jnp.float32) + c_ref[...]
    )


def _tc_out_body(m_ref, wt_ref, wb_ref, out_ref):
    for h in range(H):
        sl = slice(h * F, (h + 1) * F)
        out_ref[:, sl] = (
            jnp.dot(m_ref[:, sl], wt_ref[h], preferred_element_type=jnp.float32)
            + wb_ref[:, sl]
        )


def _splat(v, j):
    return v.at[jnp.full((16,), j, jnp.int32)].get(mode="promise_in_bounds")


def _perm(v, p):
    return v.at[p].get(mode="promise_in_bounds")


def _xor_perms():
    lanes = lax.iota(jnp.int32, 16)
    return [lanes ^ k for k in (1, 2, 4, 8)]


def _lanemax(v):
    for p in _xor_perms():
        v = jnp.maximum(v, _perm(v, p))
    return v


def _lanesum(v):
    for p in _xor_perms():
        v = v + _perm(v, p)
    return v


def _sc_gat_body(
    coff, nblk,
    adj_hbm, comb_hbm, out_hbm, idx_v, gn0, gn1, gs0, gs1, ob0, ob1,
    gsem, ssem, osem
):
    wid = lax.axis_index("s") * 2 + lax.axis_index("c")
    base = wid * (nblk * BLK)
    gns = [gn0, gn1]
    gss = [gs0, gs1]
    obs = [ob0, ob1]

    pltpu.sync_copy(adj_hbm.at[pl.ds(wid * nblk, nblk), :], idx_v)

    def issue_in(b, slot):
        pltpu.async_copy(comb_hbm.at[idx_v.at[b]], gns[slot], gsem.at[slot])
        pltpu.async_copy(
            comb_hbm.at[pl.ds(coff + base + b * BLK, BLK), :], gss[slot],
            ssem.at[slot],
        )

    issue_in(0, 0)
    issue_in(1, 1)

    @pl.loop(0, nblk, step=2)
    def _blk2(bi0):
        for par in (0, 1):
            b = bi0 + par
            slot = par
            gbase = base + b * BLK
            gn_v, gs_v, ob_v = gns[slot], gss[slot], obs[slot]
            pltpu.make_async_copy(
                comb_hbm.at[idx_v.at[b]], gn_v, gsem.at[slot]
            ).wait()
            pltpu.make_async_copy(
                comb_hbm.at[pl.ds(coff + gbase, BLK), :], gs_v, ssem.at[slot]
            ).wait()

            @pl.when(b >= 2)
            def _():
                pltpu.make_async_copy(
                    ob_v, out_hbm.at[pl.ds(gbase, BLK), :], osem.at[slot]
                ).wait()

            _node_block(gn_v, gs_v, ob_v)
            pltpu.async_copy(
                ob_v, out_hbm.at[pl.ds(gbase, BLK), :], osem.at[slot]
            )

            @pl.when(b + 2 < nblk)
            def _():
                issue_in(b + 2, slot)

    for slot in (0, 1):
        pltpu.make_async_copy(
            obs[slot], out_hbm.at[pl.ds(base, BLK), :], osem.at[slot]
        ).wait()


def _node_block(gn_v, gs_v, ob_v):
        @pl.loop(0, BLK)
        def _node(i):
            rbase = i * DEG
            lanes = lax.iota(jnp.int32, 16)
            svrow = gs_v[i, pl.ds(F, 16)]
            asv = []
            aln = []
            for h in range(H):
                col2 = jnp.full((16,), F + H + h, jnp.int32)
                s1s = _splat(svrow, h)
                s2s = _splat(svrow, H + h)
                s2n = plsc.load_gather(gn_v, [rbase + lanes, col2])
                en = s1s + s2n
                en = jnp.where(en > 0, en, 0.2 * en)
                ev = s1s + s2s
                ev = jnp.where(ev > 0, ev, 0.2 * ev)
                m = jnp.maximum(_lanemax(en), ev)
                pn = jnp.exp(en - m)
                pv = jnp.exp(ev - m)
                r = 1.0 / (_lanesum(pn) + pv)
                aln.append(pn * r)
                asv.append(pv * r)
            asp = [[_splat(aln[h], j) for j in range(DEG)] for h in range(H)]
            for ci in range(NCH):
                gs = gs_v[i, pl.ds(ci * 16, 16)]
                acc = [asv[h] * gs for h in range(H)]
                for j in range(DEG):
                    g = gn_v[rbase + j, pl.ds(ci * 16, 16)]
                    for h in range(H):
                        acc[h] = acc[h] + asp[h][j] * g
                for h in range(H):
                    ob_v[i, pl.ds(h * F + ci * 16, 16)] = acc[h]


_sc_cp = pltpu.CompilerParams(
    needs_layout_passes=False, use_tc_tiling_on_sc=False
)

NCHUNK = 2
CN = N_PAD // NCHUNK
NBLK_C = CN // NW // BLK


def _make_sc_gat(coff):
    return functools.partial(
        pl.kernel,
        compiler_params=_sc_cp,
        out_type=jax.ShapeDtypeStruct((CN, H * F), jnp.float32),
        mesh=plsc.VectorSubcoreMesh(core_axis_name="c", subcore_axis_name="s"),
        scratch_types=[
            pltpu.VMEM((NBLK_C, BLK * DEG), jnp.int32),
            pltpu.VMEM((BLK * DEG, C), jnp.float32),
            pltpu.VMEM((BLK * DEG, C), jnp.float32),
            pltpu.VMEM((BLK, C), jnp.float32),
            pltpu.VMEM((BLK, C), jnp.float32),
            pltpu.VMEM((BLK, H * F), jnp.float32),
            pltpu.VMEM((BLK, H * F), jnp.float32),
            pltpu.SemaphoreType.DMA((2,)),
            pltpu.SemaphoreType.DMA((2,)),
            pltpu.SemaphoreType.DMA((2,)),
        ],
    )(functools.partial(_sc_gat_body, coff, NBLK_C))


_sc_gats = [_make_sc_gat(c * CN) for c in range(NCHUNK)]


def kernel(adjlist, features, W, Wb, A, Ab):
    a1 = A[:, :F]
    a2 = A[:, F:]
    u1 = jnp.einsum("ho,hof->hf", a1, W)
    u2 = jnp.einsum("ho,hof->hf", a2, W)
    c1 = jnp.sum(a1 * Wb, axis=1) + Ab
    c2 = jnp.sum(a2 * Wb, axis=1)
    u = jnp.concatenate([u1.T, u2.T, jnp.zeros((F, 8), jnp.float32)], axis=1)
    cvec = jnp.concatenate([c1, c2, jnp.zeros((8,), jnp.float32)])[None, :]
    wt = jnp.swapaxes(W, 1, 2)
    wb_flat = Wb.reshape(1, H * F)

    feat_pad = jnp.pad(features, ((0, N_PAD - N), (0, 0)))
    adj2d = jnp.pad(adjlist.astype(jnp.int32), ((0, N_PAD - N), (0, 0))).reshape(
        NW * NBLK, BLK * DEG
    )

    comb = pl.pallas_call(
        _tc_pre_body,
        grid=(N_PAD // 1024,),
        in_specs=[
            pl.BlockSpec((1024, F), lambda i: (i, 0)),
            pl.BlockSpec((F, 16), lambda i: (0, 0)),
            pl.BlockSpec((1, 16), lambda i: (0, 0)),
        ],
        out_specs=pl.BlockSpec((1024, C), lambda i: (i, 0)),
        out_shape=jax.ShapeDtypeStruct((N_PAD, C), jnp.float32),
    )(feat_pad, u, cvec)

    rows_c = CN // BLK
    outs = []
    for c in range(NCHUNK):
        mixed_c = _sc_gats[c](adj2d[c * rows_c:(c + 1) * rows_c], comb)
        outs.append(
            pl.pallas_call(
                _tc_out_body,
                grid=(CN // 1024,),
                in_specs=[
                    pl.BlockSpec((1024, H * F), lambda i: (i, 0)),
                    pl.BlockSpec((H, F, F), lambda i: (0, 0, 0)),
                    pl.BlockSpec((1, H * F), lambda i: (0, 0)),
                ],
                out_specs=pl.BlockSpec((1024, H * F), lambda i: (i, 0)),
                out_shape=jax.ShapeDtypeStruct((CN, H * F), jnp.float32),
            )(mixed_c, wt, wb_flat)
        )
    return jnp.concatenate(outs, axis=0)[:N]


# speedup vs baseline: 1.5947x; 1.1589x over previous
import dataclasses
import functools

import jax
import jax.numpy as jnp
import numpy as np
from jax import lax
from jax.experimental import pallas as pl
from jax.experimental.pallas import tpu as pltpu
from jax.experimental.pallas import tpu_sc as plsc

N = 10000
DEG = 16
F = 128
H = 4
NW = 32
NPW = 320
N_PAD = NW * NPW
BLK = 16
NBLK = NPW // BLK
C = 144
NCH = F // 16


def _tc_pre_body(f_ref, u_ref, c_ref, out_ref):
    f = f_ref[...]
    out_ref[:, :F] = f
    out_ref[:, F:] = (
        jnp.dot(f, u_ref[...], preferred_element_type=jnp.float32) + c_ref[...]
    )


def _tc_out_body(m_ref, wt_ref, wb_ref, out_ref):
    for h in range(H):
        sl = slice(h * F, (h + 1) * F)
        out_ref[:, sl] = (
            jnp.dot(m_ref[:, sl], wt_ref[h], preferred_element_type=jnp.float32)
            + wb_ref[:, sl]
        )


def _splat(v, j):
    return v.at[jnp.full((16,), j, jnp.int32)].get(mode="promise_in_bounds")


def _perm(v, p):
    return v.at[p].get(mode="promise_in_bounds")


def _xor_perms():
    lanes = lax.iota(jnp.int32, 16)
    return [lanes ^ k for k in (1, 2, 4, 8)]


def _lanemax(v):
    for p in _xor_perms():
        v = jnp.maximum(v, _perm(v, p))
    return v


def _lanesum(v):
    for p in _xor_perms():
        v = v + _perm(v, p)
    return v


def _sc_gat_body(
    adj_hbm, comb_hbm, out_hbm, idx_v, gn0, gn1, gs0, gs1, ob0, ob1,
    gsem, ssem, osem
):
    wid = lax.axis_index("s") * 2 + lax.axis_index("c")
    base = wid * NPW
    gns = [gn0, gn1]
    gss = [gs0, gs1]
    obs = [ob0, ob1]

    pltpu.sync_copy(adj_hbm.at[pl.ds(wid * NBLK, NBLK), :], idx_v)

    def issue_in(b, slot):
        pltpu.async_copy(comb_hbm.at[idx_v.at[b]], gns[slot], gsem.at[slot])
        pltpu.async_copy(
            comb_hbm.at[pl.ds(base + b * BLK, BLK), :], gss[slot], ssem.at[slot]
        )

    issue_in(0, 0)
    issue_in(1, 1)

    @pl.loop(0, NBLK, step=2)
    def _blk2(bi0):
        for par in (0, 1):
            b = bi0 + par
            slot = par
            gbase = base + b * BLK
            gn_v, gs_v, ob_v = gns[slot], gss[slot], obs[slot]
            pltpu.make_async_copy(
                comb_hbm.at[idx_v.at[b]], gn_v, gsem.at[slot]
            ).wait()
            pltpu.make_async_copy(
                comb_hbm.at[pl.ds(gbase, BLK), :], gs_v, ssem.at[slot]
            ).wait()

            @pl.when(b >= 2)
            def _():
                pltpu.make_async_copy(
                    ob_v, out_hbm.at[pl.ds(gbase, BLK), :], osem.at[slot]
                ).wait()

            _node_block(gn_v, gs_v, ob_v)
            pltpu.async_copy(
                ob_v, out_hbm.at[pl.ds(gbase, BLK), :], osem.at[slot]
            )

            @pl.when(b + 2 < NBLK)
            def _():
                issue_in(b + 2, slot)

    for slot in (0, 1):
        pltpu.make_async_copy(
            obs[slot], out_hbm.at[pl.ds(base, BLK), :], osem.at[slot]
        ).wait()


def _node_block(gn_v, gs_v, ob_v):
        @pl.loop(0, BLK)
        def _node(i):
            rbase = i * DEG
            lanes = lax.iota(jnp.int32, 16)
            svrow = gs_v[i, pl.ds(F, 16)]
            asv = []
            aln = []
            for h in range(H):
                col2 = jnp.full((16,), F + H + h, jnp.int32)
                s1s = _splat(svrow, h)
                s2s = _splat(svrow, H + h)
                s2n = plsc.load_gather(gn_v, [rbase + lanes, col2])
                en = s1s + s2n
                en = jnp.where(en > 0, en, 0.2 * en)
                ev = s1s + s2s
                ev = jnp.where(ev > 0, ev, 0.2 * ev)
                m = jnp.maximum(_lanemax(en), ev)
                pn = jnp.exp(en - m)
                pv = jnp.exp(ev - m)
                r = 1.0 / (_lanesum(pn) + pv)
                aln.append(pn * r)
                asv.append(pv * r)
            asp = [[_splat(aln[h], j) for j in range(DEG)] for h in range(H)]
            for ci in range(NCH):
                gs = gs_v[i, pl.ds(ci * 16, 16)]
                acc = [asv[h] * gs for h in range(H)]
                for j in range(DEG):
                    g = gn_v[rbase + j, pl.ds(ci * 16, 16)]
                    for h in range(H):
                        acc[h] = acc[h] + asp[h][j] * g
                for h in range(H):
                    ob_v[i, pl.ds(h * F + ci * 16, 16)] = acc[h]


_sc_cp = pltpu.CompilerParams(
    needs_layout_passes=False, use_tc_tiling_on_sc=False
)

_sc_gat = functools.partial(
    pl.kernel,
    compiler_params=_sc_cp,
    out_type=jax.ShapeDtypeStruct((N_PAD, H * F), jnp.float32),
    mesh=plsc.VectorSubcoreMesh(core_axis_name="c", subcore_axis_name="s"),
    scratch_types=[
        pltpu.VMEM((NBLK, BLK * DEG), jnp.int32),
        pltpu.VMEM((BLK * DEG, C), jnp.float32),
        pltpu.VMEM((BLK * DEG, C), jnp.float32),
        pltpu.VMEM((BLK, C), jnp.float32),
        pltpu.VMEM((BLK, C), jnp.float32),
        pltpu.VMEM((BLK, H * F), jnp.float32),
        pltpu.VMEM((BLK, H * F), jnp.float32),
        pltpu.SemaphoreType.DMA((2,)),
        pltpu.SemaphoreType.DMA((2,)),
        pltpu.SemaphoreType.DMA((2,)),
    ],
)(_sc_gat_body)


def kernel(adjlist, features, W, Wb, A, Ab):
    a1 = A[:, :F]
    a2 = A[:, F:]
    u1 = jnp.einsum("ho,hof->hf", a1, W)
    u2 = jnp.einsum("ho,hof->hf", a2, W)
    c1 = jnp.sum(a1 * Wb, axis=1) + Ab
    c2 = jnp.sum(a2 * Wb, axis=1)
    u = jnp.concatenate([u1.T, u2.T, jnp.zeros((F, 8), jnp.float32)], axis=1)
    cvec = jnp.concatenate([c1, c2, jnp.zeros((8,), jnp.float32)])[None, :]
    wt = jnp.swapaxes(W, 1, 2)
    wb_flat = Wb.reshape(1, H * F)

    feat_pad = jnp.pad(features, ((0, N_PAD - N), (0, 0)))
    adj2d = jnp.pad(adjlist.astype(jnp.int32), ((0, N_PAD - N), (0, 0))).reshape(
        NW * NBLK, BLK * DEG
    )

    comb = pl.pallas_call(
        _tc_pre_body,
        grid=(N_PAD // 1024,),
        in_specs=[
            pl.BlockSpec((1024, F), lambda i: (i, 0)),
            pl.BlockSpec((F, 16), lambda i: (0, 0)),
            pl.BlockSpec((1, 16), lambda i: (0, 0)),
        ],
        out_specs=pl.BlockSpec((1024, C), lambda i: (i, 0)),
        out_shape=jax.ShapeDtypeStruct((N_PAD, C), jnp.float32),
    )(feat_pad, u, cvec)

    mixed = _sc_gat(adj2d, comb)

    out = pl.pallas_call(
        _tc_out_body,
        grid=(10,),
        in_specs=[
            pl.BlockSpec((1000, H * F), lambda i: (i, 0)),
            pl.BlockSpec((H, F, F), lambda i: (0, 0, 0)),
            pl.BlockSpec((1, H * F), lambda i: (0, 0)),
        ],
        out_specs=pl.BlockSpec((1000, H * F), lambda i: (i, 0)),
        out_shape=jax.ShapeDtypeStruct((N, H * F), jnp.float32),
    )(mixed, wt, wb_flat)
    return out
